# baseline (speedup 1.0000x reference)
import functools

import jax
import jax.numpy as jnp
from jax import lax
from jax.experimental import pallas as pl
from jax.experimental.pallas import tpu as pltpu
from jax.experimental.pallas import tpu_sc as plsc

B = 4096
D = 64
L = 16
W = 4 * D
BLK = 8 * D

_info = plsc.get_sparse_core_info()
_NC, _NS = _info.num_cores, _info.num_subcores
_NW = _NC * _NS
_BPW = B // _NW

_mesh = plsc.VectorSubcoreMesh(core_axis_name="c", subcore_axis_name="s")


@functools.partial(
    pl.kernel,
    mesh=_mesh,
    out_type=jax.ShapeDtypeStruct((B * W,), jnp.float32),
    scratch_types=[
        pltpu.VMEM((_BPW,), jnp.int32),
        pltpu.VMEM((_BPW,), jnp.int32),
        pltpu.VMEM((_BPW,), jnp.int32),
        pltpu.VMEM((_BPW,), jnp.int32),
        pltpu.SMEM((_BPW,), jnp.int32),
        pltpu.SMEM((_BPW,), jnp.int32),
        pltpu.SMEM((_BPW,), jnp.int32),
        pltpu.SMEM((_BPW,), jnp.int32),
        pltpu.VMEM((L * D,), jnp.float32),
        pltpu.VMEM((L * D,), jnp.float32),
        pltpu.VMEM((L * D,), jnp.float32),
        pltpu.VMEM((_BPW * BLK,), jnp.float32),
        pltpu.VMEM((_BPW * W,), jnp.float32),
        pltpu.SemaphoreType.DMA,
        pltpu.SemaphoreType.DMA,
        pltpu.SemaphoreType.DMA,
    ],
)
def _gather_concat(si_hbm, l0_hbm, l1_hbm, l2_hbm, zf_hbm, pf_hbm, tf_hbm,
                   bf_hbm, out_hbm, i0, i1, i2, i3, s0, s1, s2, s3,
                   ptab, ttab, btab, zbuf, rows, sema, semc, semd):
    wid = lax.axis_index("s") * _NC + lax.axis_index("c")
    base = pl.multiple_of(wid * _BPW, _BPW)

    ci0 = pltpu.async_copy(si_hbm.at[pl.ds(base, _BPW)], i0, semd)
    ci1 = pltpu.async_copy(l0_hbm.at[pl.ds(base, _BPW)], i1, semc)
    ci2 = pltpu.async_copy(l1_hbm.at[pl.ds(base, _BPW)], i2, semc)
    ci3 = pltpu.async_copy(l2_hbm.at[pl.ds(base, _BPW)], i3, semc)
    ctp = pltpu.async_copy(pf_hbm.at[pl.ds(0, L * D)], ptab, semc)
    ctt = pltpu.async_copy(tf_hbm.at[pl.ds(0, L * D)], ttab, semc)
    ctb = pltpu.async_copy(bf_hbm.at[pl.ds(0, L * D)], btab, semc)
    ci0.wait()

    for cc in range(_BPW // L):
        v0 = i0[pl.ds(cc * L, L)]
        for e in range(L):
            s0[cc * L + e] = v0[e]

    def _fire(j, _):
        s = s0[j]
        off = pl.multiple_of((s - lax.rem(s, 8)) * D, 8)
        pltpu.async_copy(zf_hbm.at[pl.ds(off, BLK)],
                         zbuf.at[pl.ds(pl.multiple_of(j * BLK, 8), BLK)],
                         sema)
        return 0

    lax.fori_loop(0, _BPW, _fire, 0, unroll=4)

    ci1.wait()
    ci2.wait()
    ci3.wait()
    ctp.wait()
    ctt.wait()
    ctb.wait()

    for cc in range(_BPW // L):
        v1 = i1[pl.ds(cc * L, L)]
        v2 = i2[pl.ds(cc * L, L)]
        v3 = i3[pl.ds(cc * L, L)]
        for e in range(L):
            s1[cc * L + e] = v1[e]
            s2[cc * L + e] = v2[e]
            s3[cc * L + e] = v3[e]

    def _cov(j, _):
        p = s1[j] * D
        t = s2[j] * D
        b = s3[j] * D
        r = j * W
        for c in range(D // L):
            rows[pl.ds(r + D + c * L, L)] = ptab[pl.ds(p + c * L, L)]
            rows[pl.ds(r + 2 * D + c * L, L)] = ttab[pl.ds(t + c * L, L)]
            rows[pl.ds(r + 3 * D + c * L, L)] = btab[pl.ds(b + c * L, L)]
        return 0

    lax.fori_loop(0, _BPW, _cov, 0)

    def _drain(j, _):
        pltpu.make_async_copy(zf_hbm.at[pl.ds(0, BLK)],
                              zbuf.at[pl.ds(0, BLK)], sema).wait()
        return 0

    lax.fori_loop(0, _BPW, _drain, 0, unroll=4)

    def _extract(j, _):
        h = lax.rem(s0[j], 8) * D
        r = j * W
        for c in range(D // L):
            rows[pl.ds(r + c * L, L)] = zbuf[pl.ds(j * BLK + h + c * L, L)]
        return 0

    lax.fori_loop(0, _BPW, _extract, 0, unroll=2)

    pltpu.sync_copy(rows, out_hbm.at[pl.ds(base * W, _BPW * W)])


def kernel(sample_indices, labels, batch_size, z_table, pert_table,
           tissue_table, batch_table):
    l0 = jnp.ravel(labels[:, 0])
    l1 = jnp.ravel(labels[:, 1])
    l2 = jnp.ravel(labels[:, 2])
    out = _gather_concat(sample_indices, l0, l1, l2,
                         jnp.ravel(z_table), jnp.ravel(pert_table),
                         jnp.ravel(tissue_table), jnp.ravel(batch_table))
    return out.reshape(B, W)


# speedup vs baseline: 1.3497x; 1.3497x over previous
import functools

import jax
import jax.numpy as jnp
from jax import lax
from jax.experimental import pallas as pl
from jax.experimental.pallas import tpu as pltpu
from jax.experimental.pallas import tpu_sc as plsc

B = 4096
D = 64
L = 16
CH = 32

_info = plsc.get_sparse_core_info()
_NC, _NS = _info.num_cores, _info.num_subcores
_NW = _NC * _NS
_BPW = B // _NW
_NCH = _BPW // CH

_mesh = plsc.VectorSubcoreMesh(core_axis_name="c", subcore_axis_name="s")


@functools.partial(
    pl.kernel,
    mesh=_mesh,
    out_type=jax.ShapeDtypeStruct((B, 4 * D), jnp.float32),
    scratch_types=[
        pltpu.VMEM((_BPW,), jnp.int32),
        pltpu.VMEM((_BPW,), jnp.int32),
        pltpu.VMEM((_BPW,), jnp.int32),
        pltpu.VMEM((_BPW,), jnp.int32),
        pltpu.SMEM((_BPW,), jnp.int32),
        pltpu.SMEM((_BPW,), jnp.int32),
        pltpu.SMEM((_BPW,), jnp.int32),
        pltpu.SMEM((_BPW,), jnp.int32),
        pltpu.VMEM((L, D), jnp.float32),
        pltpu.VMEM((L, D), jnp.float32),
        pltpu.VMEM((L, D), jnp.float32),
        pltpu.VMEM((CH, 8, D), jnp.float32),
        pltpu.VMEM((CH, 8, D), jnp.float32),
        pltpu.VMEM((_BPW, 4 * D), jnp.float32),
        pltpu.SemaphoreType.DMA,
        pltpu.SemaphoreType.DMA,
        pltpu.SemaphoreType.DMA,
        pltpu.SemaphoreType.DMA,
    ],
)
def _gather_concat(si_hbm, l0_hbm, l1_hbm, l2_hbm, z_hbm, p_hbm, t_hbm,
                   b_hbm, out_hbm, i0, i1, i2, i3, s0, s1, s2, s3,
                   ptab, ttab, btab, zba, zbb, rows, sema, semb, semc,
                   semd):
    wid = lax.axis_index("s") * _NC + lax.axis_index("c")
    base = pl.multiple_of(wid * _BPW, _BPW)

    ci0 = pltpu.async_copy(si_hbm.at[pl.ds(base, _BPW)], i0, semd)
    ci1 = pltpu.async_copy(l0_hbm.at[pl.ds(base, _BPW)], i1, semc)
    ci2 = pltpu.async_copy(l1_hbm.at[pl.ds(base, _BPW)], i2, semc)
    ci3 = pltpu.async_copy(l2_hbm.at[pl.ds(base, _BPW)], i3, semc)
    ctp = pltpu.async_copy(p_hbm.at[pl.ds(0, L)], ptab, semc)
    ctt = pltpu.async_copy(t_hbm.at[pl.ds(0, L)], ttab, semc)
    ctb = pltpu.async_copy(b_hbm.at[pl.ds(0, L)], btab, semc)
    ci0.wait()

    for cc in range(_BPW // L):
        v0 = i0[pl.ds(cc * L, L)]
        for e in range(L):
            s0[cc * L + e] = v0[e]

    bufs = (zba, zbb)
    sems = (sema, semb)

    def _fire(ch):
        buf, sem = bufs[ch % 2], sems[ch % 2]

        def body(j, _):
            s = s0[ch * CH + j]
            blk = pl.multiple_of(s - lax.rem(s, 8), 8)
            pltpu.async_copy(z_hbm.at[pl.ds(blk, 8)], buf.at[j], sem)
            return 0

        lax.fori_loop(0, CH, body, 0, unroll=4)

    def _drain(ch):
        buf, sem = bufs[ch % 2], sems[ch % 2]

        def body(j, _):
            pltpu.make_async_copy(z_hbm.at[pl.ds(0, 8)], buf.at[0],
                                  sem).wait()
            return 0

        lax.fori_loop(0, CH, body, 0, unroll=4)

    _fire(0)
    _fire(1)

    ci1.wait()
    ci2.wait()
    ci3.wait()
    ctp.wait()
    ctt.wait()
    ctb.wait()

    for cc in range(_BPW // L):
        v1 = i1[pl.ds(cc * L, L)]
        v2 = i2[pl.ds(cc * L, L)]
        v3 = i3[pl.ds(cc * L, L)]
        for e in range(L):
            s1[cc * L + e] = v1[e]
            s2[cc * L + e] = v2[e]
            s3[cc * L + e] = v3[e]

    def _cov(j, _):
        p = s1[j]
        t = s2[j]
        b = s3[j]
        for c in range(D // L):
            rows[j, pl.ds(D + c * L, L)] = ptab[p, pl.ds(c * L, L)]
            rows[j, pl.ds(2 * D + c * L, L)] = ttab[t, pl.ds(c * L, L)]
            rows[j, pl.ds(3 * D + c * L, L)] = btab[b, pl.ds(c * L, L)]
        return 0

    lax.fori_loop(0, _BPW, _cov, 0, unroll=2)

    def _extract_z(ch):
        buf = bufs[ch % 2]

        def body(j, _):
            h = lax.rem(s0[ch * CH + j], 8)
            for c in range(D // L):
                rows[ch * CH + j, pl.ds(c * L, L)] = buf[j, h,
                                                         pl.ds(c * L, L)]
            return 0

        lax.fori_loop(0, CH, body, 0, unroll=2)

    outs = []
    for ch in range(_NCH):
        _drain(ch)
        _extract_z(ch)
        if ch + 2 < _NCH:
            _fire(ch + 2)
        outs.append(pltpu.async_copy(
            rows.at[pl.ds(ch * CH, CH)],
            out_hbm.at[pl.ds(base + ch * CH, CH)], semd))
    for cp in outs:
        cp.wait()


def kernel(sample_indices, labels, batch_size, z_table, pert_table,
           tissue_table, batch_table):
    l0 = jnp.ravel(labels[:, 0])
    l1 = jnp.ravel(labels[:, 1])
    l2 = jnp.ravel(labels[:, 2])
    return _gather_concat(sample_indices, l0, l1, l2, z_table, pert_table,
                          tissue_table, batch_table)
